# Initial kernel scaffold; baseline (speedup 1.0000x reference)
import jax
import jax.numpy as jnp
from jax.experimental import pallas as pl


def kernel(node_idx, relation_idx, node_neighbor_idx, node_embed_table, relation_embed_table):
    raise NotImplementedError("write your pallas kernel here")


# speedup vs baseline: 3.7720x; 3.7720x over previous
import functools

import jax
import jax.numpy as jnp
from jax import lax
from jax.experimental import pallas as pl
from jax.experimental.pallas import tpu as pltpu
from jax.experimental.pallas import tpu_sc as plsc

_NC = 2
_NS = 16
_CHUNK = 128


@functools.lru_cache(maxsize=None)
def _make_gather(V, D, B):
    NW = _NC * _NS
    b_per_w = B // NW
    n_chunks = b_per_w // _CHUNK
    assert b_per_w * NW == B and n_chunks * _CHUNK == b_per_w
    mesh = plsc.VectorSubcoreMesh(core_axis_name="c", subcore_axis_name="s")

    @functools.partial(
        pl.kernel,
        mesh=mesh,
        compiler_params=pltpu.CompilerParams(use_tc_tiling_on_sc=False),
        out_type=[
            jax.ShapeDtypeStruct((B, D), jnp.float32),
            jax.ShapeDtypeStruct((B, D), jnp.float32),
        ],
        scratch_types=[
            pltpu.VMEM((n_chunks, _CHUNK), jnp.int32),
            pltpu.VMEM((n_chunks, _CHUNK), jnp.int32),
            pltpu.VMEM((b_per_w, D), jnp.float32),
            pltpu.VMEM((b_per_w, D), jnp.float32),
            pltpu.SemaphoreType.DMA,
        ],
    )
    def gather(table_hbm, nidx_hbm, midx_hbm, out_n, out_m,
               idx_n, idx_m, rows_n, rows_m, sem):
        wid = lax.axis_index("s") * _NC + lax.axis_index("c")
        base = wid * b_per_w
        pltpu.sync_copy(nidx_hbm.at[pl.ds(wid * n_chunks, n_chunks)], idx_n)
        pltpu.sync_copy(midx_hbm.at[pl.ds(wid * n_chunks, n_chunks)], idx_m)
        copies = []
        for j in range(n_chunks):
            copies.append(pltpu.async_copy(
                table_hbm.at[idx_n.at[j]],
                rows_n.at[pl.ds(j * _CHUNK, _CHUNK)], sem))
            copies.append(pltpu.async_copy(
                table_hbm.at[idx_m.at[j]],
                rows_m.at[pl.ds(j * _CHUNK, _CHUNK)], sem))
        for c in copies:
            c.wait()
        pltpu.sync_copy(rows_n, out_n.at[pl.ds(base, b_per_w)])
        pltpu.sync_copy(rows_m, out_m.at[pl.ds(base, b_per_w)])

    return gather


def _score_body(nrel, node_ref, nbr_ref, rel_ref, rv_ref, out_ref):
    node = node_ref[...]
    nbr = nbr_ref[...]
    rel = rel_ref[...]
    x = jnp.concatenate(
        [jnp.where(rel == r, node, 0.0) for r in range(nrel)], axis=1)
    t = lax.dot_general(x, rv_ref[...], (((1,), (0,)), ((), ())),
                        preferred_element_type=jnp.float32)
    score = jnp.sum(t * nbr, axis=1, keepdims=True)
    out_ref[...] = jax.nn.sigmoid(score)


@functools.lru_cache(maxsize=None)
def _make_score(B, D, R, Bb=1024, interpret=False):
    grid = (B // Bb,)
    return pl.pallas_call(
        functools.partial(_score_body, R),
        grid=grid,
        in_specs=[
            pl.BlockSpec((Bb, D), lambda i: (i, 0)),
            pl.BlockSpec((Bb, D), lambda i: (i, 0)),
            pl.BlockSpec((Bb, 1), lambda i: (i, 0)),
            pl.BlockSpec((R * D, D), lambda i: (0, 0)),
        ],
        out_specs=pl.BlockSpec((Bb, 1), lambda i: (i, 0)),
        out_shape=jax.ShapeDtypeStruct((B, 1), jnp.float32),
        interpret=interpret,
    )


def kernel(node_idx, relation_idx, node_neighbor_idx, node_embed_table,
           relation_embed_table):
    B = node_idx.shape[0]
    V, D = node_embed_table.shape
    R = relation_embed_table.shape[0]
    nidx2 = node_idx.astype(jnp.int32).reshape(-1, _CHUNK)
    midx2 = node_neighbor_idx.astype(jnp.int32).reshape(-1, _CHUNK)
    node_rows, nbr_rows = _make_gather(V, D, B)(
        node_embed_table, nidx2, midx2)
    rel2d = relation_idx.astype(jnp.int32).reshape(B, 1)
    rv = relation_embed_table.reshape(R * D, D)
    return _make_score(B, D, R)(node_rows, nbr_rows, rel2d, rv)


# speedup vs baseline: 4.1163x; 1.0913x over previous
import functools

import jax
import jax.numpy as jnp
from jax import lax
from jax.experimental import pallas as pl
from jax.experimental.pallas import tpu as pltpu
from jax.experimental.pallas import tpu_sc as plsc

_NC = 2
_NS = 16
_CHUNK = 128


@functools.lru_cache(maxsize=None)
def _make_gather(V, D, B):
    NW = _NC * _NS
    b_per_w = B // NW
    n_chunks = b_per_w // _CHUNK
    assert b_per_w * NW == B and n_chunks * _CHUNK == b_per_w
    mesh = plsc.VectorSubcoreMesh(core_axis_name="c", subcore_axis_name="s")

    W = 2 * D
    half = b_per_w // 2
    cph = n_chunks // 2

    @functools.partial(
        pl.kernel,
        mesh=mesh,
        compiler_params=pltpu.CompilerParams(use_tc_tiling_on_sc=False),
        out_type=[
            jax.ShapeDtypeStruct((B, W), jnp.float32),
            jax.ShapeDtypeStruct((B, W), jnp.float32),
        ],
        scratch_types=[
            pltpu.VMEM((n_chunks, _CHUNK), jnp.int32),
            pltpu.VMEM((n_chunks, _CHUNK), jnp.int32),
            pltpu.VMEM((b_per_w, D), jnp.float32),
            pltpu.VMEM((b_per_w, D), jnp.float32),
            pltpu.SemaphoreType.DMA,
        ],
    )
    def gather(table_hbm, nidx_hbm, midx_hbm, out_n, out_m,
               idx_n, idx_m, rows_n, rows_m, sem):
        wid = lax.axis_index("s") * _NC + lax.axis_index("c")
        base = wid * b_per_w
        pltpu.sync_copy(nidx_hbm.at[pl.ds(wid * n_chunks, n_chunks)], idx_n)
        pltpu.sync_copy(midx_hbm.at[pl.ds(wid * n_chunks, n_chunks)], idx_m)
        copies = []
        for j in range(n_chunks):
            dst = pl.ds(j * _CHUNK, _CHUNK)
            copies.append(pltpu.async_copy(
                table_hbm.at[idx_n.at[j]], rows_n.at[dst], sem))
            copies.append(pltpu.async_copy(
                table_hbm.at[idx_m.at[j]], rows_m.at[dst], sem))
        for c in copies:
            c.wait()
        pltpu.sync_copy(rows_n, out_n.at[pl.ds(base, b_per_w), pl.ds(0, D)])
        pltpu.sync_copy(rows_m, out_m.at[pl.ds(base, b_per_w), pl.ds(0, D)])

    return gather


def _score_body(nrel, node_ref, nbr_ref, rel_ref, rv_ref, out_ref):
    D = rv_ref.shape[1]
    node = node_ref[:, :D]
    nbr = nbr_ref[:, :D]
    rel = rel_ref[...]
    x = jnp.concatenate(
        [jnp.where(rel == r, node, 0.0) for r in range(nrel)], axis=1)
    t = lax.dot_general(x, rv_ref[...], (((1,), (0,)), ((), ())),
                        preferred_element_type=jnp.float32)
    score = jnp.sum(t * nbr, axis=1, keepdims=True)
    out_ref[...] = jax.nn.sigmoid(score)


@functools.lru_cache(maxsize=None)
def _make_score(B, D, R, Bb=1024, interpret=False):
    grid = (B // Bb,)
    return pl.pallas_call(
        functools.partial(_score_body, R),
        grid=grid,
        in_specs=[
            pl.BlockSpec((Bb, 2 * D), lambda i: (i, 0)),
            pl.BlockSpec((Bb, 2 * D), lambda i: (i, 0)),
            pl.BlockSpec((Bb, 1), lambda i: (i, 0)),
            pl.BlockSpec((R * D, D), lambda i: (0, 0)),
        ],
        out_specs=pl.BlockSpec((Bb, 1), lambda i: (i, 0)),
        out_shape=jax.ShapeDtypeStruct((B, 1), jnp.float32),
        interpret=interpret,
    )


def kernel(node_idx, relation_idx, node_neighbor_idx, node_embed_table,
           relation_embed_table):
    B = node_idx.shape[0]
    V, D = node_embed_table.shape
    R = relation_embed_table.shape[0]
    nidx2 = node_idx.astype(jnp.int32).reshape(-1, _CHUNK)
    midx2 = node_neighbor_idx.astype(jnp.int32).reshape(-1, _CHUNK)
    node_rows, nbr_rows = _make_gather(V, D, B)(
        node_embed_table, nidx2, midx2)
    rel2d = relation_idx.astype(jnp.int32).reshape(B, 1)
    rv = relation_embed_table.reshape(R * D, D)
    return _make_score(B, D, R)(node_rows, nbr_rows, rel2d, rv)
